# baseline (speedup 1.0000x reference)
import functools

import jax
import jax.numpy as jnp
from jax import lax
from jax.experimental import pallas as pl
from jax.experimental.pallas import tpu as pltpu
from jax.experimental.pallas import tpu_sc as plsc

N = 1024
L = 16


@jax.jit
def _sc_build_adj(edge_index, ea):
    E = ea.shape[0]
    e_pad = ((E + L - 1) // L) * L
    info = plsc.get_sparse_core_info()
    nc, ns = info.num_cores, info.num_subcores
    nw = nc * ns
    rows_per_w = N // nw

    mesh = plsc.VectorSubcoreMesh(core_axis_name="c", subcore_axis_name="s")

    @functools.partial(
        pl.kernel,
        mesh=mesh,
        compiler_params=pltpu.CompilerParams(
            needs_layout_passes=False,
            use_tc_tiling_on_sc=False,
            skip_device_barrier=True,
        ),
        out_type=jax.ShapeDtypeStruct((N, N), jnp.float32),
        scratch_types=[
            pltpu.VMEM((e_pad,), jnp.int32),
            pltpu.VMEM((e_pad,), jnp.int32),
            pltpu.VMEM((e_pad,), jnp.float32),
            pltpu.VMEM((rows_per_w, N), jnp.float32),
            pltpu.SemaphoreType.DMA,
        ],
    )
    def sc_kernel(ei_hbm, ea_hbm, a_hbm, row_v, col_v, ea_v, blk_v, sem):
        wid = lax.axis_index("s") * nc + lax.axis_index("c")
        lo = wid * rows_per_w

        if e_pad != E:
            col_v[pl.ds(e_pad - L, L)] = jnp.full((L,), N, jnp.int32)
        cp1 = pltpu.async_copy(ei_hbm.at[0], row_v.at[pl.ds(0, E)], sem)
        cp2 = pltpu.async_copy(ei_hbm.at[1], col_v.at[pl.ds(0, E)], sem)
        cp3 = pltpu.async_copy(ea_hbm, ea_v.at[pl.ds(0, E)], sem)

        npl = N // L

        @plsc.parallel_loop(0, rows_per_w * npl, unroll=4)
        def zero_body(i):
            blk_v[i // npl, pl.ds((i % npl) * L, L)] = jnp.zeros((L,), jnp.float32)

        cp1.wait()
        cp2.wait()
        cp3.wait()

        @plsc.parallel_loop(0, e_pad // L, unroll=2)
        def edge_body(i):
            r = row_v[pl.ds(i * L, L)]
            c = col_v[pl.ds(i * L, L)]
            w = ea_v[pl.ds(i * L, L)]
            msk = (c >= lo) & (c < lo + rows_per_w)
            plsc.store_scatter(blk_v, [c - lo, r], w, mask=msk)

        pltpu.sync_copy(blk_v, a_hbm.at[pl.ds(lo, rows_per_w)])

    return sc_kernel(edge_index, ea)


def _h_body(x_ref, w_ref, h_ref):
    h_ref[0] = lax.dot_general(
        x_ref[0].astype(jnp.bfloat16),
        w_ref[...].astype(jnp.bfloat16),
        (((1,), (1,)), ((), ())),
        preferred_element_type=jnp.float32,
    ).astype(jnp.bfloat16)


def _tc_h(x, W):
    B, n, d_in = x.shape
    d_out = W.shape[0]
    return pl.pallas_call(
        _h_body,
        grid=(B,),
        in_specs=[
            pl.BlockSpec((1, n, d_in), lambda i: (i, 0, 0)),
            pl.BlockSpec((d_out, d_in), lambda i: (0, 0)),
        ],
        out_specs=pl.BlockSpec((1, n, d_out), lambda i: (i, 0, 0)),
        out_shape=jax.ShapeDtypeStruct((B, n, d_out), jnp.bfloat16),
    )(x, W)


def _main_body(h_ref, bias_ref, a_ref, out_ref, dinv_scr, abf_scr):
    @pl.when(pl.program_id(0) == 0)
    def _():
        a = a_ref[...]
        dinv_scr[...] = lax.rsqrt(1.0 + jnp.sum(a, axis=1, keepdims=True))
        abf_scr[...] = a.astype(jnp.bfloat16)

    dinv_col = dinv_scr[...]
    h = h_ref[0].astype(jnp.float32)
    hs = (h * dinv_col).astype(jnp.bfloat16)
    m = jnp.dot(abf_scr[...], hs, preferred_element_type=jnp.float32)
    out_ref[0] = m * dinv_col + h * (dinv_col * dinv_col) + bias_ref[...]


@jax.jit
def _tc_gcn(x, W, bvec, a):
    B, n, d_in = x.shape
    d_out = W.shape[0]
    h = _tc_h(x, W)
    return pl.pallas_call(
        _main_body,
        grid=(B,),
        in_specs=[
            pl.BlockSpec((1, n, d_out), lambda b: (b, 0, 0)),
            pl.BlockSpec((1, d_out), lambda b: (0, 0)),
            pl.BlockSpec((n, n), lambda b: (0, 0)),
        ],
        out_specs=pl.BlockSpec((1, n, d_out), lambda b: (b, 0, 0)),
        out_shape=jax.ShapeDtypeStruct((B, n, d_out), jnp.float32),
        scratch_shapes=[
            pltpu.VMEM((n, 1), jnp.float32),
            pltpu.VMEM((n, n), jnp.bfloat16),
        ],
    )(h, bvec.reshape(1, d_out), a)


def kernel(x, W, b, edge_index, edge_attr):
    a = _sc_build_adj(edge_index.astype(jnp.int32), edge_attr.astype(jnp.float32))
    return _tc_gcn(x, W, b, a)


# speedup vs baseline: 1.1364x; 1.1364x over previous
import functools

import jax
import jax.numpy as jnp
from jax import lax
from jax.experimental import pallas as pl
from jax.experimental.pallas import tpu as pltpu
from jax.experimental.pallas import tpu_sc as plsc

N = 1024
L = 16


@jax.jit
def _sc_build_adj(row, col, ea):
    E = ea.shape[0]
    e_pad = ((E + L - 1) // L) * L
    info = plsc.get_sparse_core_info()
    nc, ns = info.num_cores, info.num_subcores
    nw = nc * ns
    rows_per_w = N // nw

    mesh = plsc.VectorSubcoreMesh(core_axis_name="c", subcore_axis_name="s")

    @functools.partial(
        pl.kernel,
        mesh=mesh,
        compiler_params=pltpu.CompilerParams(
            needs_layout_passes=False,
            skip_device_barrier=True,
        ),
        out_type=jax.ShapeDtypeStruct((N, N), jnp.float32),
        scratch_types=[
            pltpu.VMEM((e_pad,), jnp.int32),
            pltpu.VMEM((e_pad,), jnp.int32),
            pltpu.VMEM((e_pad,), jnp.float32),
            pltpu.VMEM((rows_per_w, N), jnp.float32),
            pltpu.SemaphoreType.DMA,
        ],
    )
    def sc_kernel(row_hbm, col_hbm, ea_hbm, a_hbm, row_v, col_v, ea_v, blk_v, sem):
        wid = lax.axis_index("s") * nc + lax.axis_index("c")
        lo = wid * rows_per_w

        if e_pad != E:
            col_v[pl.ds(e_pad - L, L)] = jnp.full((L,), N, jnp.int32)
        cp1 = pltpu.async_copy(row_hbm, row_v.at[pl.ds(0, E)], sem)
        cp2 = pltpu.async_copy(col_hbm, col_v.at[pl.ds(0, E)], sem)
        cp3 = pltpu.async_copy(ea_hbm, ea_v.at[pl.ds(0, E)], sem)

        npl = N // L

        @plsc.parallel_loop(0, rows_per_w * npl, unroll=4)
        def zero_body(i):
            blk_v[i // npl, pl.ds((i % npl) * L, L)] = jnp.zeros((L,), jnp.float32)

        cp1.wait()
        cp2.wait()
        cp3.wait()

        @plsc.parallel_loop(0, e_pad // L, unroll=2)
        def edge_body(i):
            r = row_v[pl.ds(i * L, L)]
            c = col_v[pl.ds(i * L, L)]
            w = ea_v[pl.ds(i * L, L)]
            msk = (c >= lo) & (c < lo + rows_per_w)
            plsc.store_scatter(blk_v, [c - lo, r], w, mask=msk)

        pltpu.sync_copy(blk_v, a_hbm.at[pl.ds(lo, rows_per_w)])

    return sc_kernel(row, col, ea)


def _h_body(x_ref, w_ref, h_ref):
    wb = w_ref[...].astype(jnp.bfloat16)
    for j in range(x_ref.shape[0]):
        h_ref[j] = lax.dot_general(
            x_ref[j].astype(jnp.bfloat16),
            wb,
            (((1,), (1,)), ((), ())),
            preferred_element_type=jnp.float32,
        ).astype(jnp.bfloat16)


def _tc_h(x, W):
    B, n, d_in = x.shape
    d_out = W.shape[0]
    bb = 2
    return pl.pallas_call(
        _h_body,
        grid=(B // bb,),
        in_specs=[
            pl.BlockSpec((bb, n, d_in), lambda i: (i, 0, 0)),
            pl.BlockSpec((d_out, d_in), lambda i: (0, 0)),
        ],
        out_specs=pl.BlockSpec((bb, n, d_out), lambda i: (i, 0, 0)),
        out_shape=jax.ShapeDtypeStruct((B, n, d_out), jnp.bfloat16),
    )(x, W)


def _main_body(h_ref, bias_ref, a_ref, out_ref, dinv_scr, abf_scr):
    @pl.when(pl.program_id(0) == 0)
    def _():
        a = a_ref[...]
        dinv_scr[...] = lax.rsqrt(1.0 + jnp.sum(a, axis=1, keepdims=True))
        abf_scr[...] = a.astype(jnp.bfloat16)

    dinv_col = dinv_scr[...]
    h = h_ref[0].astype(jnp.float32)
    hs = (h * dinv_col).astype(jnp.bfloat16)
    m = jnp.dot(abf_scr[...], hs, preferred_element_type=jnp.float32)
    out_ref[0] = m * dinv_col + h * (dinv_col * dinv_col) + bias_ref[...]


@jax.jit
def _tc_gcn(x, W, bvec, a):
    B, n, d_in = x.shape
    d_out = W.shape[0]
    h = _tc_h(x, W)
    return pl.pallas_call(
        _main_body,
        grid=(B,),
        in_specs=[
            pl.BlockSpec((1, n, d_out), lambda b: (b, 0, 0)),
            pl.BlockSpec((1, d_out), lambda b: (0, 0)),
            pl.BlockSpec((n, n), lambda b: (0, 0)),
        ],
        out_specs=pl.BlockSpec((1, n, d_out), lambda b: (b, 0, 0)),
        out_shape=jax.ShapeDtypeStruct((B, n, d_out), jnp.float32),
        scratch_shapes=[
            pltpu.VMEM((n, 1), jnp.float32),
            pltpu.VMEM((n, n), jnp.bfloat16),
        ],
    )(h, bvec.reshape(1, d_out), a)


def kernel(x, W, b, edge_index, edge_attr):
    ei = edge_index.astype(jnp.int32)
    a = _sc_build_adj(ei[0], ei[1], edge_attr.astype(jnp.float32))
    return _tc_gcn(x, W, b, a)


# speedup vs baseline: 1.2383x; 1.0897x over previous
import functools

import jax
import jax.numpy as jnp
from jax import lax
from jax.experimental import pallas as pl
from jax.experimental.pallas import tpu as pltpu
from jax.experimental.pallas import tpu_sc as plsc

N = 1024
L = 16


@jax.jit
def _sc_build_adj(row, col, ea):
    E = ea.shape[0]
    e_pad = ((E + L - 1) // L) * L
    info = plsc.get_sparse_core_info()
    nc, ns = info.num_cores, info.num_subcores
    nw = nc * ns
    rows_per_w = N // nw

    mesh = plsc.VectorSubcoreMesh(core_axis_name="c", subcore_axis_name="s")

    @functools.partial(
        pl.kernel,
        mesh=mesh,
        compiler_params=pltpu.CompilerParams(
            needs_layout_passes=False,
            skip_device_barrier=True,
        ),
        out_type=jax.ShapeDtypeStruct((N, N), jnp.float32),
        scratch_types=[
            pltpu.VMEM((e_pad,), jnp.int32),
            pltpu.VMEM((e_pad,), jnp.int32),
            pltpu.VMEM((e_pad,), jnp.float32),
            pltpu.VMEM((rows_per_w, N), jnp.float32),
            pltpu.SemaphoreType.DMA,
        ],
    )
    def sc_kernel(row_hbm, col_hbm, ea_hbm, a_hbm, row_v, col_v, ea_v, blk_v, sem):
        wid = lax.axis_index("s") * nc + lax.axis_index("c")
        lo = wid * rows_per_w

        if e_pad != E:
            col_v[pl.ds(e_pad - L, L)] = jnp.full((L,), N, jnp.int32)
        cp1 = pltpu.async_copy(row_hbm, row_v.at[pl.ds(0, E)], sem)
        cp2 = pltpu.async_copy(col_hbm, col_v.at[pl.ds(0, E)], sem)
        cp3 = pltpu.async_copy(ea_hbm, ea_v.at[pl.ds(0, E)], sem)

        npl = N // L

        @plsc.parallel_loop(0, rows_per_w * npl, unroll=4)
        def zero_body(i):
            blk_v[i // npl, pl.ds((i % npl) * L, L)] = jnp.zeros((L,), jnp.float32)

        cp1.wait()
        cp2.wait()
        cp3.wait()

        @plsc.parallel_loop(0, e_pad // L, unroll=2)
        def edge_body(i):
            r = row_v[pl.ds(i * L, L)]
            c = col_v[pl.ds(i * L, L)]
            w = ea_v[pl.ds(i * L, L)]
            msk = (c >= lo) & (c < lo + rows_per_w)
            plsc.store_scatter(blk_v, [c - lo, r], w, mask=msk)

        pltpu.sync_copy(blk_v, a_hbm.at[pl.ds(lo, rows_per_w)])

    return sc_kernel(row, col, ea)


def _h_body(x_ref, w_ref, h_ref):
    wb = w_ref[...].astype(jnp.bfloat16)
    for j in range(x_ref.shape[0]):
        h_ref[j] = lax.dot_general(
            x_ref[j].astype(jnp.bfloat16),
            wb,
            (((1,), (1,)), ((), ())),
            preferred_element_type=jnp.float32,
        ).astype(jnp.bfloat16)


def _tc_h(x, W):
    B, n, d_in = x.shape
    d_out = W.shape[0]
    bb = 4
    return pl.pallas_call(
        _h_body,
        grid=(B // bb,),
        in_specs=[
            pl.BlockSpec((bb, n, d_in), lambda i: (i, 0, 0)),
            pl.BlockSpec((d_out, d_in), lambda i: (0, 0)),
        ],
        out_specs=pl.BlockSpec((bb, n, d_out), lambda i: (i, 0, 0)),
        out_shape=jax.ShapeDtypeStruct((B, n, d_out), jnp.bfloat16),
    )(x, W)


def _main_body(h_ref, bias_ref, a_ref, out_ref, dinv_scr, abf_scr):
    @pl.when(pl.program_id(0) == 0)
    def _():
        a = a_ref[...]
        dinv_scr[...] = lax.rsqrt(1.0 + jnp.sum(a, axis=1, keepdims=True))
        abf_scr[...] = a.astype(jnp.bfloat16)

    dinv_col = dinv_scr[...]
    abf = abf_scr[...]
    bias = bias_ref[...]
    for j in range(h_ref.shape[0]):
        h = h_ref[j].astype(jnp.float32)
        hs = (h * dinv_col).astype(jnp.bfloat16)
        m = jnp.dot(abf, hs, preferred_element_type=jnp.float32)
        out_ref[j] = m * dinv_col + h * (dinv_col * dinv_col) + bias


@jax.jit
def _tc_gcn(x, W, bvec, a):
    B, n, d_in = x.shape
    d_out = W.shape[0]
    h = _tc_h(x, W)
    bb = 2
    return pl.pallas_call(
        _main_body,
        grid=(B // bb,),
        in_specs=[
            pl.BlockSpec((bb, n, d_out), lambda b: (b, 0, 0)),
            pl.BlockSpec((1, d_out), lambda b: (0, 0)),
            pl.BlockSpec((n, n), lambda b: (0, 0)),
        ],
        out_specs=pl.BlockSpec((bb, n, d_out), lambda b: (b, 0, 0)),
        out_shape=jax.ShapeDtypeStruct((B, n, d_out), jnp.float32),
        scratch_shapes=[
            pltpu.VMEM((n, 1), jnp.float32),
            pltpu.VMEM((n, n), jnp.bfloat16),
        ],
    )(h, bvec.reshape(1, d_out), a)


def kernel(x, W, b, edge_index, edge_attr):
    ei = edge_index.astype(jnp.int32)
    a = _sc_build_adj(ei[0], ei[1], edge_attr.astype(jnp.float32))
    return _tc_gcn(x, W, b, a)


# speedup vs baseline: 1.2831x; 1.0361x over previous
import functools

import jax
import jax.numpy as jnp
from jax import lax
from jax.experimental import pallas as pl
from jax.experimental.pallas import tpu as pltpu
from jax.experimental.pallas import tpu_sc as plsc

N = 1024
L = 16


@jax.jit
def _sc_build_adj(row, col, ea):
    E = ea.shape[0]
    e_pad = ((E + L - 1) // L) * L
    info = plsc.get_sparse_core_info()
    nc, ns = info.num_cores, info.num_subcores
    nw = nc * ns
    rows_per_w = N // nw

    mesh = plsc.VectorSubcoreMesh(core_axis_name="c", subcore_axis_name="s")

    @functools.partial(
        pl.kernel,
        mesh=mesh,
        compiler_params=pltpu.CompilerParams(
            needs_layout_passes=False,
            skip_device_barrier=True,
        ),
        out_type=jax.ShapeDtypeStruct((N, N), jnp.float32),
        scratch_types=[
            pltpu.VMEM((e_pad,), jnp.int32),
            pltpu.VMEM((e_pad,), jnp.int32),
            pltpu.VMEM((e_pad,), jnp.float32),
            pltpu.VMEM((rows_per_w, N), jnp.float32),
            pltpu.SemaphoreType.DMA,
        ],
    )
    def sc_kernel(row_hbm, col_hbm, ea_hbm, a_hbm, row_v, col_v, ea_v, blk_v, sem):
        wid = lax.axis_index("s") * nc + lax.axis_index("c")
        lo = wid * rows_per_w

        if e_pad != E:
            col_v[pl.ds(e_pad - L, L)] = jnp.full((L,), N, jnp.int32)
        cp1 = pltpu.async_copy(row_hbm, row_v.at[pl.ds(0, E)], sem)
        cp2 = pltpu.async_copy(col_hbm, col_v.at[pl.ds(0, E)], sem)
        cp3 = pltpu.async_copy(ea_hbm, ea_v.at[pl.ds(0, E)], sem)

        npl = N // L

        @plsc.parallel_loop(0, rows_per_w * npl, unroll=4)
        def zero_body(i):
            blk_v[i // npl, pl.ds((i % npl) * L, L)] = jnp.zeros((L,), jnp.float32)

        cp1.wait()
        cp2.wait()
        cp3.wait()

        @plsc.parallel_loop(0, e_pad // L, unroll=2)
        def edge_body(i):
            r = row_v[pl.ds(i * L, L)]
            c = col_v[pl.ds(i * L, L)]
            w = ea_v[pl.ds(i * L, L)]
            msk = (c >= lo) & (c < lo + rows_per_w)
            plsc.store_scatter(blk_v, [c - lo, r], w, mask=msk)

        pltpu.sync_copy(blk_v, a_hbm.at[pl.ds(lo, rows_per_w)])

    return sc_kernel(row, col, ea)


def _h_body(x_ref, w_ref, h_ref):
    wb = w_ref[...].astype(jnp.bfloat16)
    for j in range(x_ref.shape[0]):
        h_ref[j] = lax.dot_general(
            x_ref[j].astype(jnp.bfloat16),
            wb,
            (((1,), (1,)), ((), ())),
            preferred_element_type=jnp.float32,
        ).astype(jnp.bfloat16)


def _tc_h(x, W):
    B, n, d_in = x.shape
    d_out = W.shape[0]
    bb = 8
    return pl.pallas_call(
        _h_body,
        grid=(B // bb,),
        in_specs=[
            pl.BlockSpec((bb, n, d_in), lambda i: (i, 0, 0)),
            pl.BlockSpec((d_out, d_in), lambda i: (0, 0)),
        ],
        out_specs=pl.BlockSpec((bb, n, d_out), lambda i: (i, 0, 0)),
        out_shape=jax.ShapeDtypeStruct((B, n, d_out), jnp.bfloat16),
    )(x, W)


def _main_body(h_ref, bias_ref, a_ref, out_ref, dinv_scr, abf_scr):
    @pl.when(pl.program_id(0) == 0)
    def _():
        a = a_ref[...]
        dinv_scr[...] = lax.rsqrt(1.0 + jnp.sum(a, axis=1, keepdims=True))
        abf_scr[...] = a.astype(jnp.bfloat16)

    dinv_col = dinv_scr[...]
    abf = abf_scr[...]
    bias = bias_ref[...]
    for j in range(h_ref.shape[0]):
        h = h_ref[j].astype(jnp.float32)
        hs = (h * dinv_col).astype(jnp.bfloat16)
        m = jnp.dot(abf, hs, preferred_element_type=jnp.float32)
        out_ref[j] = m * dinv_col + h * (dinv_col * dinv_col) + bias


@jax.jit
def _tc_gcn(x, W, bvec, a):
    B, n, d_in = x.shape
    d_out = W.shape[0]
    h = _tc_h(x, W)
    bb = 4
    return pl.pallas_call(
        _main_body,
        grid=(B // bb,),
        in_specs=[
            pl.BlockSpec((bb, n, d_out), lambda b: (b, 0, 0)),
            pl.BlockSpec((1, d_out), lambda b: (0, 0)),
            pl.BlockSpec((n, n), lambda b: (0, 0)),
        ],
        out_specs=pl.BlockSpec((bb, n, d_out), lambda b: (b, 0, 0)),
        out_shape=jax.ShapeDtypeStruct((B, n, d_out), jnp.float32),
        scratch_shapes=[
            pltpu.VMEM((n, 1), jnp.float32),
            pltpu.VMEM((n, n), jnp.bfloat16),
        ],
    )(h, bvec.reshape(1, d_out), a)


def kernel(x, W, b, edge_index, edge_attr):
    ei = edge_index.astype(jnp.int32)
    a = _sc_build_adj(ei[0], ei[1], edge_attr.astype(jnp.float32))
    return _tc_gcn(x, W, b, a)
